# Initial kernel scaffold; baseline (speedup 1.0000x reference)
import jax
import jax.numpy as jnp
from jax.experimental import pallas as pl


def kernel(amounts, emb):
    raise NotImplementedError("write your pallas kernel here")


# speedup vs baseline: 2.3096x; 2.3096x over previous
import functools
import jax
import jax.numpy as jnp
from jax import lax
from jax.experimental import pallas as pl
from jax.experimental.pallas import tpu as pltpu
from jax.experimental.pallas import tpu_sc as plsc

_NUM_BUCKETS = 12
_EMB_DIM = 32
_BOUNDS = (1.0, 2.0, 5.0, 10.0, 20.0, 50.0, 100.0, 200.0, 500.0, 1000.0, 2000.0)

_NC = 2
_NS = 16
_NW = _NC * _NS
_L = 16


@functools.lru_cache(maxsize=None)
def _build_sc_call(n):
    per_w = n // _NW
    c = 3200
    while per_w % c:
        c //= 2
    iters = per_w // c
    groups = c // _L

    @functools.partial(
        pl.kernel,
        mesh=plsc.VectorSubcoreMesh(core_axis_name="c", subcore_axis_name="s"),
        out_type=jax.ShapeDtypeStruct((n * _EMB_DIM,), jnp.float32),
        scratch_types=[
            pltpu.VMEM((_NUM_BUCKETS * _EMB_DIM,), jnp.float32),
            pltpu.VMEM((c,), jnp.float32),
            pltpu.VMEM((c * _EMB_DIM,), jnp.float32),
        ],
        compiler_params=pltpu.CompilerParams(needs_layout_passes=False),
    )
    def sc_call(amounts_hbm, emb_hbm, out_hbm, emb_v, amt_v, out_v):
        wid = lax.axis_index("s") * _NC + lax.axis_index("c")
        base = wid * per_w
        pltpu.sync_copy(emb_hbm, emb_v)
        jbase = lax.iota(jnp.int32, _L) * _EMB_DIM

        def chunk(i, carry):
            off = base + i * c
            pltpu.sync_copy(amounts_hbm.at[pl.ds(off, c)], amt_v)

            def group(g, carry2):
                a = amt_v[pl.ds(g * _L, _L)]
                acc = jnp.zeros((_L,), jnp.int32)
                for b in _BOUNDS:
                    acc = acc + jnp.where(a >= b, _EMB_DIM, 0)
                sbase = g * (_L * _EMB_DIM)
                sidx = jbase + sbase
                for k in range(_EMB_DIM):
                    vals = plsc.load_gather(emb_v, [acc + k])
                    plsc.store_scatter(out_v, [sidx + k], vals)
                return carry2

            lax.fori_loop(0, groups, group, 0)
            pltpu.sync_copy(out_v, out_hbm.at[pl.ds(off * _EMB_DIM, c * _EMB_DIM)])
            return carry

        lax.fori_loop(0, iters, chunk, 0)

    return sc_call


def kernel(amounts, emb):
    bsz, seq = amounts.shape
    n = bsz * seq
    out = _build_sc_call(n)(
        amounts.reshape(n), emb.reshape(_NUM_BUCKETS * _EMB_DIM)
    )
    return out.reshape(bsz, seq, _EMB_DIM)
